# baseline (speedup 1.0000x reference)
import functools

import jax
import jax.numpy as jnp
import numpy as np
from jax import lax
from jax.experimental import pallas as pl
from jax.experimental.pallas import tpu as pltpu
from jax.experimental.pallas import tpu_sc as plsc

_NUM_CORES = 2
_NUM_SUBCORES = 16
_NUM_WORKERS = _NUM_CORES * _NUM_SUBCORES

_NUM_MODAL = 3


def _build_labels(modal_lens, n_emb_rows):
    modal_different = n_emb_rows == 2 * _NUM_MODAL
    labels = []
    for i, length in enumerate(modal_lens):
        labels.append(i + _NUM_MODAL if modal_different else i)
        labels.extend([i] * (length - 1))
    return np.asarray(labels, dtype=np.int32)


@functools.lru_cache(maxsize=None)
def _make_sc_call(batch, total_rows, d_model, labels_key, n_emb_rows):
    labels_seq = np.asarray(labels_key, dtype=np.int32)
    labels_flat = np.tile(labels_seq, batch)
    n_rows = batch * total_rows
    assert n_rows % _NUM_WORKERS == 0
    rows_per_w = n_rows // _NUM_WORKERS
    chunk = 32
    assert rows_per_w % chunk == 0
    n_chunks = rows_per_w // chunk

    rep_flat = np.empty_like(labels_flat)
    uniform = True
    for w in range(_NUM_WORKERS):
        lo = w * rows_per_w
        const = labels_flat[lo + 1] if rows_per_w > 1 else labels_flat[lo]
        rep_flat[lo:lo + rows_per_w] = const
        if not np.all(labels_flat[lo + chunk:lo + rows_per_w] == const):
            uniform = False

    mesh = plsc.VectorSubcoreMesh(core_axis_name="c", subcore_axis_name="s")

    @functools.partial(
        pl.kernel,
        mesh=mesh,
        out_type=jax.ShapeDtypeStruct((n_rows, d_model), jnp.float32),
        scratch_types=[
            pltpu.VMEM((chunk,), jnp.int32),
            pltpu.VMEM((chunk,), jnp.int32),
            pltpu.VMEM((chunk, d_model), jnp.float32),
            pltpu.VMEM((chunk, d_model), jnp.float32),
            pltpu.SemaphoreType.DMA,
            pltpu.SemaphoreType.DMA,
            pltpu.SemaphoreType.DMA,
        ],
    )
    def sc_call(emb_hbm, lab_hbm, rep_hbm, out_hbm,
                idx_a, idx_b, buf_a, buf_b, sem_a, sem_b, wsem):
        wid = lax.axis_index("s") * _NUM_CORES + lax.axis_index("c")
        base = wid * rows_per_w
        if uniform:
            pltpu.sync_copy(lab_hbm.at[pl.ds(base, chunk)], idx_a)
            pltpu.sync_copy(rep_hbm.at[pl.ds(base, chunk)], idx_b)
            ga = pltpu.async_copy(emb_hbm.at[idx_a], buf_a, sem_a)
            gb = pltpu.async_copy(emb_hbm.at[idx_b], buf_b, sem_b)
            ga.wait()
            writes = [pltpu.async_copy(buf_a, out_hbm.at[pl.ds(base, chunk)], wsem)]
            gb.wait()
            for c in range(1, n_chunks):
                writes.append(pltpu.async_copy(
                    buf_b, out_hbm.at[pl.ds(base + c * chunk, chunk)], wsem))
            for wr in writes:
                wr.wait()
        else:
            idx = (idx_a, idx_b)
            buf = (buf_a, buf_b)
            sem = (sem_a, sem_b)
            gathers = [None, None]
            writes = [None, None]
            for c in range(n_chunks):
                p = c % 2
                if writes[p] is not None:
                    writes[p].wait()
                pltpu.sync_copy(lab_hbm.at[pl.ds(base + c * chunk, chunk)], idx[p])
                gathers[p] = pltpu.async_copy(emb_hbm.at[idx[p]], buf[p], sem[p])
                gathers[p].wait()
                writes[p] = pltpu.async_copy(
                    buf[p], out_hbm.at[pl.ds(base + c * chunk, chunk)], wsem)
            for wr in writes:
                if wr is not None:
                    wr.wait()

    return sc_call, labels_flat, rep_flat


def kernel(modal_feat_0, modal_feat_1, modal_feat_2, modal_emb):
    modal_lens = (modal_feat_0.shape[1], modal_feat_1.shape[1],
                  modal_feat_2.shape[1])
    batch = modal_feat_0.shape[0]
    d_model = modal_emb.shape[1]
    n_emb_rows = modal_emb.shape[0]
    labels_seq = _build_labels(modal_lens, n_emb_rows)
    total_rows = int(labels_seq.shape[0])
    sc_call, labels_flat, rep_flat = _make_sc_call(
        batch, total_rows, d_model, tuple(int(x) for x in labels_seq),
        n_emb_rows)
    out_flat = sc_call(modal_emb,
                       jnp.asarray(labels_flat),
                       jnp.asarray(rep_flat))
    return out_flat.reshape(batch, total_rows, d_model)
